# Initial kernel scaffold; baseline (speedup 1.0000x reference)
import jax
import jax.numpy as jnp
from jax.experimental import pallas as pl


def kernel(token_ids, embedding_table):
    raise NotImplementedError("write your pallas kernel here")


# speedup vs baseline: 1.4769x; 1.4769x over previous
import functools

import jax
import jax.numpy as jnp
from jax import lax
from jax.experimental import pallas as pl
from jax.experimental.pallas import tpu as pltpu
from jax.experimental.pallas import tpu_sc as plsc

_NC = 2
_NS = 16
_NW = _NC * _NS

_B = 4096
_S = 200
_D = 32
_R = _B * _S
_RPW = _R // _NW
_CH = 128
_NCH = _RPW // _CH
_K = 8
_BLK = _K * _CH
_NBLK = _NCH // _K


def _gather_body(idx_hbm, table_hbm, out_hbm, idx_v, rows_v, sem):
    wid = lax.axis_index("s") * _NC + lax.axis_index("c")
    ibase = wid * _NCH
    obase = wid * _RPW

    pltpu.sync_copy(idx_hbm.at[pl.ds(ibase, _NCH)], idx_v)

    def blk_body(blk, carry):
        waits = []
        for k in range(_K):
            waits.append(
                pltpu.async_copy(
                    table_hbm.at[idx_v.at[blk * _K + k]],
                    rows_v.at[pl.ds(k * _CH, _CH)],
                    sem,
                )
            )
        for w in waits:
            w.wait()
        pltpu.sync_copy(rows_v, out_hbm.at[pl.ds(obase + blk * _BLK, _BLK)])
        return carry

    lax.fori_loop(0, _NBLK, blk_body, 0)


@functools.partial(
    pl.kernel,
    out_type=jax.ShapeDtypeStruct((_R, _D), jnp.float32),
    mesh=plsc.VectorSubcoreMesh(core_axis_name="c", subcore_axis_name="s"),
    scratch_types=[
        pltpu.VMEM((_NCH, _CH), jnp.int32),
        pltpu.VMEM((_BLK, _D), jnp.float32),
        pltpu.SemaphoreType.DMA,
    ],
    compiler_params=pltpu.CompilerParams(use_tc_tiling_on_sc=False),
)
def _gather_call(idx_hbm, table_hbm, out_hbm, idx_v, rows_v, sem):
    _gather_body(idx_hbm, table_hbm, out_hbm, idx_v, rows_v, sem)


@jax.jit
def kernel(token_ids, embedding_table):
    idx = token_ids.reshape(_R // _CH, _CH).astype(jnp.int32)
    out = _gather_call(idx, embedding_table)
    return out.reshape(_B, _S, _D)


# speedup vs baseline: 1.4954x; 1.0125x over previous
import functools

import jax
import jax.numpy as jnp
from jax import lax
from jax.experimental import pallas as pl
from jax.experimental.pallas import tpu as pltpu
from jax.experimental.pallas import tpu_sc as plsc

_NC = 2
_NS = 16
_NW = _NC * _NS

_B = 4096
_S = 200
_D = 32
_R = _B * _S
_RPW = _R // _NW
_CH = 128
_NCH = _RPW // _CH
_K = 8
_BLK = _K * _CH
_NBLK = _NCH // _K


def _gather_body(idx_hbm, table_hbm, out_hbm, idx_v, rows_v, sem, wsem):
    wid = lax.axis_index("s") * _NC + lax.axis_index("c")
    ibase = wid * _NCH
    obase = wid * _RPW

    pltpu.sync_copy(idx_hbm.at[pl.ds(ibase, _NCH)], idx_v)

    def blk_body(blk, carry):
        slot = lax.rem(blk, 2)
        rows = rows_v.at[slot]

        @pl.when(blk >= 2)
        def _():
            pltpu.make_async_copy(out_hbm.at[pl.ds(obase, _BLK)], rows, wsem).wait()

        waits = []
        for k in range(_K):
            waits.append(
                pltpu.async_copy(
                    table_hbm.at[idx_v.at[blk * _K + k]],
                    rows.at[pl.ds(k * _CH, _CH)],
                    sem,
                )
            )
        for w in waits:
            w.wait()
        pltpu.async_copy(rows, out_hbm.at[pl.ds(obase + blk * _BLK, _BLK)], wsem)
        return carry

    lax.fori_loop(0, _NBLK, blk_body, 0)

    for slot in range(2):
        pltpu.make_async_copy(
            out_hbm.at[pl.ds(obase, _BLK)], rows_v.at[slot], wsem
        ).wait()


@functools.partial(
    pl.kernel,
    out_type=jax.ShapeDtypeStruct((_R, _D), jnp.float32),
    mesh=plsc.VectorSubcoreMesh(core_axis_name="c", subcore_axis_name="s"),
    scratch_types=[
        pltpu.VMEM((_NCH, _CH), jnp.int32),
        pltpu.VMEM((2, _BLK, _D), jnp.float32),
        pltpu.SemaphoreType.DMA,
        pltpu.SemaphoreType.DMA,
    ],
    compiler_params=pltpu.CompilerParams(use_tc_tiling_on_sc=False),
)
def _gather_call(idx_hbm, table_hbm, out_hbm, idx_v, rows_v, sem, wsem):
    _gather_body(idx_hbm, table_hbm, out_hbm, idx_v, rows_v, sem, wsem)


@jax.jit
def kernel(token_ids, embedding_table):
    idx = token_ids.reshape(_R // _CH, _CH).astype(jnp.int32)
    out = _gather_call(idx, embedding_table)
    return out.reshape(_B, _S, _D)
